# Initial kernel scaffold; baseline (speedup 1.0000x reference)
import jax
import jax.numpy as jnp
from jax.experimental import pallas as pl


def kernel(inputs, adj, W1, b1, W2, b2, gcW, gcb, We2p, be2p):
    raise NotImplementedError("write your pallas kernel here")


# speedup vs baseline: 1.1404x; 1.1404x over previous
import functools

import jax
import jax.numpy as jnp
from jax.experimental import pallas as pl
from jax.experimental.pallas import tpu as pltpu

N = 4096
F = 512
H = 512
NOUT = 128

RB_A = 512
RB_G = 512


def _encode_support(x_ref, w1_ref, b1_ref, w2_ref, b2_ref, gw_ref, s_ref):
    h = jnp.dot(x_ref[...], w1_ref[...], preferred_element_type=jnp.float32)
    h = jnp.maximum(h + b1_ref[...], 0.0)
    h = jnp.dot(h, w2_ref[...], preferred_element_type=jnp.float32)
    h = jnp.maximum(h + b2_ref[...], 0.0)
    s_ref[...] = jnp.dot(h, gw_ref[...], preferred_element_type=jnp.float32)


def _agg_support(adj_ref, s_ref, gcb_ref, gw_ref, out_ref):
    agg = jnp.dot(adj_ref[...], s_ref[...], preferred_element_type=jnp.float32)
    h = jnp.maximum(agg + gcb_ref[...], 0.0)
    out_ref[...] = jnp.dot(h, gw_ref[...], preferred_element_type=jnp.float32)


def _agg_final(adj_ref, s_ref, gcb_ref, wz_ref, bz_ref, emb_ref, z_ref, acc_ref):
    i = pl.program_id(0)
    agg = jnp.dot(adj_ref[...], s_ref[...], preferred_element_type=jnp.float32)
    h = jnp.maximum(agg + gcb_ref[...], 0.0)
    emb_ref[...] = h
    colsum = jnp.sum(h, axis=0, keepdims=True)

    @pl.when(i == 0)
    def _init():
        acc_ref[...] = colsum

    @pl.when(i > 0)
    def _accum():
        acc_ref[...] = acc_ref[...] + colsum

    @pl.when(i == pl.num_programs(0) - 1)
    def _decode():
        mean = acc_ref[...] * (1.0 / N)
        z_ref[...] = (
            jnp.dot(mean, wz_ref[...], preferred_element_type=jnp.float32)
            + bz_ref[...]
        )


def kernel(inputs, adj, W1, b1, W2, b2, gcW, gcb, We2p, be2p):
    x2d = inputs.reshape(N, F)
    b1r = b1.reshape(1, H)
    b2r = b2.reshape(1, H)
    gcbr = gcb.reshape(1, H)
    be2pr = be2p.reshape(1, NOUT)

    full = lambda *shape: pl.BlockSpec(shape, lambda i: (0,) * len(shape))

    s1 = pl.pallas_call(
        _encode_support,
        grid=(N // RB_A,),
        in_specs=[
            pl.BlockSpec((RB_A, F), lambda i: (i, 0)),
            full(F, H), full(1, H), full(H, H), full(1, H), full(H, H),
        ],
        out_specs=pl.BlockSpec((RB_A, H), lambda i: (i, 0)),
        out_shape=jax.ShapeDtypeStruct((N, H), jnp.float32),
    )(x2d, W1, b1r, W2, b2r, gcW)

    s2 = pl.pallas_call(
        _agg_support,
        grid=(N // RB_G,),
        in_specs=[
            pl.BlockSpec((RB_G, N), lambda i: (i, 0)),
            full(N, H), full(1, H), full(H, H),
        ],
        out_specs=pl.BlockSpec((RB_G, H), lambda i: (i, 0)),
        out_shape=jax.ShapeDtypeStruct((N, H), jnp.float32),
    )(adj, s1, gcbr, gcW)

    emb, z = pl.pallas_call(
        _agg_final,
        grid=(N // RB_G,),
        in_specs=[
            pl.BlockSpec((RB_G, N), lambda i: (i, 0)),
            full(N, H), full(1, H), full(H, NOUT), full(1, NOUT),
        ],
        out_specs=[
            pl.BlockSpec((RB_G, H), lambda i: (i, 0)),
            pl.BlockSpec((1, NOUT), lambda i: (0, 0)),
        ],
        out_shape=[
            jax.ShapeDtypeStruct((N, H), jnp.float32),
            jax.ShapeDtypeStruct((1, NOUT), jnp.float32),
        ],
        scratch_shapes=[pltpu.VMEM((1, H), jnp.float32)],
    )(adj, s2, gcbr, We2p, be2pr)

    return (emb.reshape(1, N, H), z)


# speedup vs baseline: 1.1440x; 1.0031x over previous
import functools

import jax
import jax.numpy as jnp
from jax.experimental import pallas as pl
from jax.experimental.pallas import tpu as pltpu

N = 4096
F = 512
H = 512
NOUT = 128

RB_A = 512
RB_G = 512


def _bdot(a, b):
    return jnp.dot(a.astype(jnp.bfloat16), b.astype(jnp.bfloat16),
                   preferred_element_type=jnp.float32)


def _encode_support(x_ref, w1_ref, b1_ref, w2_ref, b2_ref, gw_ref, s_ref):
    h = _bdot(x_ref[...], w1_ref[...])
    h = jnp.maximum(h + b1_ref[...], 0.0)
    h = _bdot(h, w2_ref[...])
    h = jnp.maximum(h + b2_ref[...], 0.0)
    s_ref[...] = _bdot(h, gw_ref[...])


def _agg_support(adj_ref, s_ref, gcb_ref, gw_ref, out_ref):
    agg = _bdot(adj_ref[...], s_ref[...])
    h = jnp.maximum(agg + gcb_ref[...], 0.0)
    out_ref[...] = _bdot(h, gw_ref[...])


def _agg_final(adj_ref, s_ref, gcb_ref, wz_ref, bz_ref, emb_ref, z_ref, acc_ref):
    i = pl.program_id(0)
    agg = _bdot(adj_ref[...], s_ref[...])
    h = jnp.maximum(agg + gcb_ref[...], 0.0)
    emb_ref[...] = h
    colsum = jnp.sum(h, axis=0, keepdims=True)

    @pl.when(i == 0)
    def _init():
        acc_ref[...] = colsum

    @pl.when(i > 0)
    def _accum():
        acc_ref[...] = acc_ref[...] + colsum

    @pl.when(i == pl.num_programs(0) - 1)
    def _decode():
        mean = acc_ref[...] * (1.0 / N)
        z_ref[...] = (
            jnp.dot(mean, wz_ref[...], preferred_element_type=jnp.float32)
            + bz_ref[...]
        )


def kernel(inputs, adj, W1, b1, W2, b2, gcW, gcb, We2p, be2p):
    x2d = inputs.reshape(N, F)
    b1r = b1.reshape(1, H)
    b2r = b2.reshape(1, H)
    gcbr = gcb.reshape(1, H)
    be2pr = be2p.reshape(1, NOUT)

    full = lambda *shape: pl.BlockSpec(shape, lambda i: (0,) * len(shape))

    s1 = pl.pallas_call(
        _encode_support,
        grid=(N // RB_A,),
        in_specs=[
            pl.BlockSpec((RB_A, F), lambda i: (i, 0)),
            full(F, H), full(1, H), full(H, H), full(1, H), full(H, H),
        ],
        out_specs=pl.BlockSpec((RB_A, H), lambda i: (i, 0)),
        out_shape=jax.ShapeDtypeStruct((N, H), jnp.float32),
    )(x2d, W1, b1r, W2, b2r, gcW)

    s2 = pl.pallas_call(
        _agg_support,
        grid=(N // RB_G,),
        in_specs=[
            pl.BlockSpec((RB_G, N), lambda i: (i, 0)),
            full(N, H), full(1, H), full(H, H),
        ],
        out_specs=pl.BlockSpec((RB_G, H), lambda i: (i, 0)),
        out_shape=jax.ShapeDtypeStruct((N, H), jnp.float32),
    )(adj, s1, gcbr, gcW)

    emb, z = pl.pallas_call(
        _agg_final,
        grid=(N // RB_G,),
        in_specs=[
            pl.BlockSpec((RB_G, N), lambda i: (i, 0)),
            full(N, H), full(1, H), full(H, NOUT), full(1, NOUT),
        ],
        out_specs=[
            pl.BlockSpec((RB_G, H), lambda i: (i, 0)),
            pl.BlockSpec((1, NOUT), lambda i: (0, 0)),
        ],
        out_shape=[
            jax.ShapeDtypeStruct((N, H), jnp.float32),
            jax.ShapeDtypeStruct((1, NOUT), jnp.float32),
        ],
        scratch_shapes=[pltpu.VMEM((1, H), jnp.float32)],
    )(adj, s2, gcbr, We2p, be2pr)

    return (emb.reshape(1, N, H), z)
